# baseline (speedup 1.0000x reference)
import jax
import jax.numpy as jnp
from jax.experimental import pallas as pl
from jax.experimental.pallas import tpu as pltpu

_BL = 64
_NRING = 48
_LAG = 24


def kernel(x, x_len):
    B, L, D = x.shape
    nj = L // _BL
    NB = B * nj
    xlen = x_len.astype(jnp.int32)

    def body(xlen_ref, x_hbm, o_hbm, bufs, zbuf, rsem, wsem):
        zbuf[...] = jnp.zeros_like(zbuf)

        def oblock(k):
            b, j = divmod(k, nj)
            return o_hbm.at[b, pl.ds(j * _BL, _BL), :]

        def xblock(k):
            b, j = divmod(k, nj)
            return x_hbm.at[b, pl.ds(j * _BL, _BL), :]

        def nvalid(k):
            b, j = divmod(k, nj)
            return xlen_ref[b] - j * _BL

        def wsem_wait(r):
            pltpu.make_async_copy(zbuf, oblock(0), wsem.at[r]).wait()

        def retire(kr):
            rr = kr % _NRING
            nv = nvalid(kr)

            @pl.when(nv > 0)
            def _():
                pltpu.make_async_copy(
                    xblock(kr), bufs.at[rr], rsem.at[rr]
                ).wait()

                @pl.when(nv < _BL)
                def _():
                    row = jax.lax.broadcasted_iota(jnp.int32, (_BL, D), 0)
                    bufs[rr] = jnp.where(row < nv, bufs[rr], 0.0)

                pltpu.make_async_copy(
                    bufs.at[rr], oblock(kr), wsem.at[rr]
                ).start()

            @pl.when(nv <= 0)
            def _():
                pltpu.make_async_copy(zbuf, oblock(kr), wsem.at[rr]).start()

        for k in range(NB):
            r = k % _NRING
            if k >= _NRING:
                wsem_wait(r)

            @pl.when(nvalid(k) > 0)
            def _(k=k, r=r):
                pltpu.make_async_copy(xblock(k), bufs.at[r], rsem.at[r]).start()

            if k >= _LAG:
                retire(k - _LAG)

        for kr in range(NB - _LAG, NB):
            retire(kr)

        for r in range(_NRING):
            wsem_wait(r)

    grid_spec = pltpu.PrefetchScalarGridSpec(
        num_scalar_prefetch=1,
        grid=(1,),
        in_specs=[pl.BlockSpec(memory_space=pl.ANY)],
        out_specs=pl.BlockSpec(memory_space=pl.ANY),
        scratch_shapes=[
            pltpu.VMEM((_NRING, _BL, D), jnp.float32),
            pltpu.VMEM((_BL, D), jnp.float32),
            pltpu.SemaphoreType.DMA((_NRING,)),
            pltpu.SemaphoreType.DMA((_NRING,)),
        ],
    )
    return pl.pallas_call(
        body,
        grid_spec=grid_spec,
        out_shape=jax.ShapeDtypeStruct((B, L, D), x.dtype),
    )(xlen, x)


# speedup vs baseline: 1.0230x; 1.0230x over previous
import jax
import jax.numpy as jnp
from jax.experimental import pallas as pl
from jax.experimental.pallas import tpu as pltpu

_BL = 128
_NRING = 32
_LAG = 16
_SUB = 32


def kernel(x, x_len):
    B, L, D = x.shape
    nj = L // _BL
    NB = B * nj
    xlen = x_len.astype(jnp.int32)

    def body(xlen_ref, x_hbm, o_hbm, bufs, zbuf, rsem, wsem):
        zbuf[...] = jnp.zeros_like(zbuf)

        def oblock(k):
            b, j = divmod(k, nj)
            return o_hbm.at[b, pl.ds(j * _BL, _BL), :]

        def xblock(k):
            b, j = divmod(k, nj)
            return x_hbm.at[b, pl.ds(j * _BL, _BL), :]

        def xsub(k, s):
            b, j = divmod(k, nj)
            return x_hbm.at[b, pl.ds(j * _BL + s * _SUB, _SUB), :]

        def nvalid(k):
            b, j = divmod(k, nj)
            return xlen_ref[b] - j * _BL

        def wsem_wait(r):
            pltpu.make_async_copy(zbuf, oblock(0), wsem.at[r]).wait()

        def retire(kr):
            rr = kr % _NRING
            nv = nvalid(kr)

            @pl.when(nv >= _BL)
            def _():
                pltpu.make_async_copy(
                    xblock(kr), bufs.at[rr], rsem.at[rr]
                ).wait()
                pltpu.make_async_copy(
                    bufs.at[rr], oblock(kr), wsem.at[rr]
                ).start()

            @pl.when((nv > 0) & (nv < _BL))
            def _():
                for s in range(_BL // _SUB):
                    @pl.when(nv > s * _SUB)
                    def _(s=s):
                        pltpu.make_async_copy(
                            xsub(kr, s),
                            bufs.at[rr, pl.ds(s * _SUB, _SUB), :],
                            rsem.at[rr],
                        ).wait()

                row = jax.lax.broadcasted_iota(jnp.int32, (_BL, D), 0)
                bufs[rr] = jnp.where(row < nv, bufs[rr], 0.0)
                pltpu.make_async_copy(
                    bufs.at[rr], oblock(kr), wsem.at[rr]
                ).start()

            @pl.when(nv <= 0)
            def _():
                pltpu.make_async_copy(zbuf, oblock(kr), wsem.at[rr]).start()

        for k in range(NB):
            r = k % _NRING
            if k >= _NRING:
                wsem_wait(r)

            nv = nvalid(k)

            @pl.when(nv >= _BL)
            def _(k=k, r=r):
                pltpu.make_async_copy(xblock(k), bufs.at[r], rsem.at[r]).start()

            @pl.when((nv > 0) & (nv < _BL))
            def _(k=k, r=r, nv=nv):
                for s in range(_BL // _SUB):
                    @pl.when(nv > s * _SUB)
                    def _(s=s):
                        pltpu.make_async_copy(
                            xsub(k, s),
                            bufs.at[r, pl.ds(s * _SUB, _SUB), :],
                            rsem.at[r],
                        ).start()

            if k >= _LAG:
                retire(k - _LAG)

        for kr in range(NB - _LAG, NB):
            retire(kr)

        for r in range(_NRING):
            wsem_wait(r)

    grid_spec = pltpu.PrefetchScalarGridSpec(
        num_scalar_prefetch=1,
        grid=(1,),
        in_specs=[pl.BlockSpec(memory_space=pl.ANY)],
        out_specs=pl.BlockSpec(memory_space=pl.ANY),
        scratch_shapes=[
            pltpu.VMEM((_NRING, _BL, D), jnp.float32),
            pltpu.VMEM((_BL, D), jnp.float32),
            pltpu.SemaphoreType.DMA((_NRING,)),
            pltpu.SemaphoreType.DMA((_NRING,)),
        ],
    )
    return pl.pallas_call(
        body,
        grid_spec=grid_spec,
        out_shape=jax.ShapeDtypeStruct((B, L, D), x.dtype),
    )(xlen, x)


# speedup vs baseline: 1.0263x; 1.0032x over previous
import jax
import jax.numpy as jnp
from jax.experimental import pallas as pl
from jax.experimental.pallas import tpu as pltpu

_BL = 128
_NRING = 40
_LAG = 24
_SUB = 32


def kernel(x, x_len):
    B, L, D = x.shape
    nj = L // _BL
    NB = B * nj
    xlen = x_len.astype(jnp.int32)

    def body(xlen_ref, x_hbm, o_hbm, bufs, zbuf, rsem, wsem):
        zbuf[...] = jnp.zeros_like(zbuf)

        def oblock(k):
            b, j = divmod(k, nj)
            return o_hbm.at[b, pl.ds(j * _BL, _BL), :]

        def xblock(k):
            b, j = divmod(k, nj)
            return x_hbm.at[b, pl.ds(j * _BL, _BL), :]

        def xsub(k, s):
            b, j = divmod(k, nj)
            return x_hbm.at[b, pl.ds(j * _BL + s * _SUB, _SUB), :]

        def nvalid(k):
            b, j = divmod(k, nj)
            return xlen_ref[b] - j * _BL

        def wsem_wait(r):
            pltpu.make_async_copy(zbuf, oblock(0), wsem.at[r]).wait()

        def retire(kr):
            rr = kr % _NRING
            nv = nvalid(kr)

            @pl.when(nv >= _BL)
            def _():
                pltpu.make_async_copy(
                    xblock(kr), bufs.at[rr], rsem.at[rr]
                ).wait()
                pltpu.make_async_copy(
                    bufs.at[rr], oblock(kr), wsem.at[rr]
                ).start()

            @pl.when((nv > 0) & (nv < _BL))
            def _():
                for s in range(_BL // _SUB):
                    @pl.when(nv > s * _SUB)
                    def _(s=s):
                        pltpu.make_async_copy(
                            xsub(kr, s),
                            bufs.at[rr, pl.ds(s * _SUB, _SUB), :],
                            rsem.at[rr],
                        ).wait()

                row = jax.lax.broadcasted_iota(jnp.int32, (_BL, D), 0)
                bufs[rr] = jnp.where(row < nv, bufs[rr], 0.0)
                pltpu.make_async_copy(
                    bufs.at[rr], oblock(kr), wsem.at[rr]
                ).start()

            @pl.when(nv <= 0)
            def _():
                pltpu.make_async_copy(zbuf, oblock(kr), wsem.at[rr]).start()

        for k in range(NB):
            r = k % _NRING
            if k >= _NRING:
                wsem_wait(r)

            nv = nvalid(k)

            @pl.when(nv >= _BL)
            def _(k=k, r=r):
                pltpu.make_async_copy(xblock(k), bufs.at[r], rsem.at[r]).start()

            @pl.when((nv > 0) & (nv < _BL))
            def _(k=k, r=r, nv=nv):
                for s in range(_BL // _SUB):
                    @pl.when(nv > s * _SUB)
                    def _(s=s):
                        pltpu.make_async_copy(
                            xsub(k, s),
                            bufs.at[r, pl.ds(s * _SUB, _SUB), :],
                            rsem.at[r],
                        ).start()

            if k >= _LAG:
                retire(k - _LAG)

        for kr in range(NB - _LAG, NB):
            retire(kr)

        for r in range(_NRING):
            wsem_wait(r)

    grid_spec = pltpu.PrefetchScalarGridSpec(
        num_scalar_prefetch=1,
        grid=(1,),
        in_specs=[pl.BlockSpec(memory_space=pl.ANY)],
        out_specs=pl.BlockSpec(memory_space=pl.ANY),
        scratch_shapes=[
            pltpu.VMEM((_NRING, _BL, D), jnp.float32),
            pltpu.VMEM((_BL, D), jnp.float32),
            pltpu.SemaphoreType.DMA((_NRING,)),
            pltpu.SemaphoreType.DMA((_NRING,)),
        ],
    )
    return pl.pallas_call(
        body,
        grid_spec=grid_spec,
        out_shape=jax.ShapeDtypeStruct((B, L, D), x.dtype),
    )(xlen, x)


# speedup vs baseline: 1.0296x; 1.0032x over previous
import jax
import jax.numpy as jnp
from jax.experimental import pallas as pl
from jax.experimental.pallas import tpu as pltpu

_BL = 128
_NRING = 40
_LAG = 24
_SUB = 16


def kernel(x, x_len):
    B, L, D = x.shape
    nj = L // _BL
    NB = B * nj
    xlen = x_len.astype(jnp.int32)

    def body(xlen_ref, x_hbm, o_hbm, bufs, zbuf, rsem, wsem):
        zbuf[...] = jnp.zeros_like(zbuf)

        def oblock(k):
            b, j = divmod(k, nj)
            return o_hbm.at[b, pl.ds(j * _BL, _BL), :]

        def xblock(k):
            b, j = divmod(k, nj)
            return x_hbm.at[b, pl.ds(j * _BL, _BL), :]

        def xsub(k, s):
            b, j = divmod(k, nj)
            return x_hbm.at[b, pl.ds(j * _BL + s * _SUB, _SUB), :]

        def nvalid(k):
            b, j = divmod(k, nj)
            return xlen_ref[b] - j * _BL

        def wsem_wait(r):
            pltpu.make_async_copy(zbuf, oblock(0), wsem.at[r]).wait()

        def retire(kr):
            rr = kr % _NRING
            nv = nvalid(kr)

            @pl.when(nv >= _BL)
            def _():
                pltpu.make_async_copy(
                    xblock(kr), bufs.at[rr], rsem.at[rr]
                ).wait()
                pltpu.make_async_copy(
                    bufs.at[rr], oblock(kr), wsem.at[rr]
                ).start()

            @pl.when((nv > 0) & (nv < _BL))
            def _():
                for s in range(_BL // _SUB):
                    @pl.when(nv > s * _SUB)
                    def _(s=s):
                        pltpu.make_async_copy(
                            xsub(kr, s),
                            bufs.at[rr, pl.ds(s * _SUB, _SUB), :],
                            rsem.at[rr],
                        ).wait()

                row = jax.lax.broadcasted_iota(jnp.int32, (_BL, D), 0)
                bufs[rr] = jnp.where(row < nv, bufs[rr], 0.0)
                pltpu.make_async_copy(
                    bufs.at[rr], oblock(kr), wsem.at[rr]
                ).start()

            @pl.when(nv <= 0)
            def _():
                pltpu.make_async_copy(zbuf, oblock(kr), wsem.at[rr]).start()

        for k in range(NB):
            r = k % _NRING
            if k >= _NRING:
                wsem_wait(r)

            nv = nvalid(k)

            @pl.when(nv >= _BL)
            def _(k=k, r=r):
                pltpu.make_async_copy(xblock(k), bufs.at[r], rsem.at[r]).start()

            @pl.when((nv > 0) & (nv < _BL))
            def _(k=k, r=r, nv=nv):
                for s in range(_BL // _SUB):
                    @pl.when(nv > s * _SUB)
                    def _(s=s):
                        pltpu.make_async_copy(
                            xsub(k, s),
                            bufs.at[r, pl.ds(s * _SUB, _SUB), :],
                            rsem.at[r],
                        ).start()

            if k >= _LAG:
                retire(k - _LAG)

        for kr in range(NB - _LAG, NB):
            retire(kr)

        for r in range(_NRING):
            wsem_wait(r)

    grid_spec = pltpu.PrefetchScalarGridSpec(
        num_scalar_prefetch=1,
        grid=(1,),
        in_specs=[pl.BlockSpec(memory_space=pl.ANY)],
        out_specs=pl.BlockSpec(memory_space=pl.ANY),
        scratch_shapes=[
            pltpu.VMEM((_NRING, _BL, D), jnp.float32),
            pltpu.VMEM((_BL, D), jnp.float32),
            pltpu.SemaphoreType.DMA((_NRING,)),
            pltpu.SemaphoreType.DMA((_NRING,)),
        ],
    )
    return pl.pallas_call(
        body,
        grid_spec=grid_spec,
        out_shape=jax.ShapeDtypeStruct((B, L, D), x.dtype),
    )(xlen, x)
